# Initial kernel scaffold; baseline (speedup 1.0000x reference)
import jax
import jax.numpy as jnp
from jax.experimental import pallas as pl


def kernel(x, pe, edge_index, edge_attr, batch, params):
    raise NotImplementedError("write your pallas kernel here")


# speedup vs baseline: 3.5473x; 3.5473x over previous
import functools

import jax
import jax.numpy as jnp
from jax import lax
from jax.experimental import pallas as pl
from jax.experimental.pallas import tpu as pltpu
from jax.experimental.pallas import tpu_sc as plsc

N = 10000
C = 128
G = 64
PE = 8
WL = 20
H = 4
DH = C // H
E = 320000
NINV = 1.0 / N
EPS = 1e-5
F32 = jnp.float32

NC = 2
NS = 16
NW = NC * NS
CH = 128
NCHUNK = E // CH
CPT = (NCHUNK + NW - 1) // NW
RA = 624
RLAST = N - RA * (NS - 1)

BQ = 400
NBQ = N // BQ


def _sc_aggr_body(h_hbm, src_hbm, dst_hbm, attr_hbm, etab_hbm, zeros_hbm,
                  out_hbm, sidx_v, didx_v, aidx_v, rows_v, ea_v, sem, sem2,
                  etab_sh, aggr_sh):
    c = lax.axis_index("c")
    s = lax.axis_index("s")
    wid = s * NC + c

    @pl.when(s == 0)
    def _():
        pltpu.sync_copy(etab_hbm, etab_sh)

    @pl.when(s < NS - 1)
    def _():
        pltpu.sync_copy(zeros_hbm.at[pl.ds(s * RA, RA)],
                        aggr_sh.at[pl.ds(s * RA, RA)])

    @pl.when(s == NS - 1)
    def _():
        pltpu.sync_copy(zeros_hbm.at[pl.ds(RA * (NS - 1), RLAST)],
                        aggr_sh.at[pl.ds(RA * (NS - 1), RLAST)])

    plsc.subcore_barrier()

    def chunk_body(i, carry):
        cid = i * NW + wid

        @pl.when(cid < NCHUNK)
        def _():
            base = cid * CH
            pltpu.sync_copy(src_hbm.at[pl.ds(base, CH)], sidx_v)
            pltpu.sync_copy(dst_hbm.at[pl.ds(base, CH)], didx_v)
            pltpu.sync_copy(attr_hbm.at[pl.ds(base, CH)], aidx_v)
            cp1 = pltpu.async_copy(h_hbm.at[sidx_v], rows_v, sem)
            cp2 = pltpu.async_copy(etab_sh.at[aidx_v], ea_v, sem2)
            cp1.wait()
            cp2.wait()

            def edge_body(e, carry2):
                for j in range(C // 16):
                    sl = pl.ds(j * 16, 16)
                    rows_v[e, sl] = jnp.maximum(rows_v[e, sl] + ea_v[e, sl],
                                                0.0)
                return carry2

            lax.fori_loop(0, CH, edge_body, 0)
            pltpu.sync_copy(rows_v, aggr_sh.at[didx_v], add=True)

        return carry

    lax.fori_loop(0, CPT, chunk_body, 0)
    plsc.subcore_barrier()

    @pl.when(s < NS - 1)
    def _():
        pltpu.sync_copy(aggr_sh.at[pl.ds(s * RA, RA)],
                        out_hbm.at[c, pl.ds(s * RA, RA)])

    @pl.when(s == NS - 1)
    def _():
        pltpu.sync_copy(aggr_sh.at[pl.ds(RA * (NS - 1), RLAST)],
                        out_hbm.at[c, pl.ds(RA * (NS - 1), RLAST)])


def _sc_aggr(h, src, dst, attr, etab, zeros):
    mesh = plsc.VectorSubcoreMesh(core_axis_name="c", subcore_axis_name="s",
                                  num_cores=NC, num_subcores=NS)
    f = pl.kernel(
        _sc_aggr_body,
        out_type=jax.ShapeDtypeStruct((NC, N, C), F32),
        mesh=mesh,
        scratch_types=[
            pltpu.VMEM((CH,), jnp.int32),
            pltpu.VMEM((CH,), jnp.int32),
            pltpu.VMEM((CH,), jnp.int32),
            pltpu.VMEM((CH, C), F32),
            pltpu.VMEM((CH, C), F32),
            pltpu.SemaphoreType.DMA,
            pltpu.SemaphoreType.DMA,
            pltpu.VMEM_SHARED((4, C), F32),
            pltpu.VMEM_SHARED((N, C), F32),
        ],
    )
    return f(h, src, dst, attr, etab, zeros)


def _bn_rows(x, g, b):
    m = jnp.sum(x, axis=0, keepdims=True) * NINV
    ex2 = jnp.sum(x * x, axis=0, keepdims=True) * NINV
    v = ex2 - m * m
    return (x - m) * lax.rsqrt(v + EPS) * g + b


def _embed_body(xcol_ref, pe_ref, wn_ref, wp_ref, bc_ref, g_ref, b_ref,
                out_ref):
    pe = pe_ref[...]
    pen = _bn_rows(pe, g_ref[...], b_ref[...])
    onehot = (xcol_ref[...] ==
              lax.broadcasted_iota(jnp.int32, (1, 28), 1)).astype(F32)
    out_ref[...] = (
        jnp.dot(onehot, wn_ref[...], preferred_element_type=F32)
        + jnp.dot(pen, wp_ref[...], preferred_element_type=F32)
        + bc_ref[...])


def _gine_body(h_ref, ap_ref, w1_ref, b1_ref, w2_ref, b2_ref, g_ref, bb_ref,
               out_ref):
    h = h_ref[...]
    z = h + ap_ref[0] + ap_ref[1]
    z = jnp.maximum(jnp.dot(z, w1_ref[...], preferred_element_type=F32)
                    + b1_ref[...], 0.0)
    z = jnp.dot(z, w2_ref[...], preferred_element_type=F32) + b2_ref[...]
    out_ref[...] = _bn_rows(z + h, g_ref[...], bb_ref[...])


def _attn_body(h_ref, qlo_ref, qhi_ref, lo_ref, hi_ref, wq_ref, wk_ref,
               wv_ref, bq_ref, bk_ref, bv_ref, wo_ref, bo_ref, out_ref):
    def qblock(i, carry):
        r0 = i * BQ
        hq = h_ref[pl.ds(r0, BQ), :]
        qlo = qlo_ref[pl.ds(r0, BQ), :]
        qhi = qhi_ref[pl.ds(r0, BQ), :]
        j0 = lo_ref[i] // BQ
        j1 = (hi_ref[i] + BQ - 1) // BQ
        o = jnp.zeros((BQ, C), F32)
        for hh in range(H):
            qh = (jnp.dot(hq, wq_ref[hh], preferred_element_type=F32)
                  + bq_ref[hh])

            def kblock(j, ca):
                mx, l, acc = ca
                ks = j * BQ
                hk = h_ref[pl.ds(ks, BQ), :]
                kh = (jnp.dot(hk, wk_ref[hh], preferred_element_type=F32)
                      + bk_ref[hh])
                vh = (jnp.dot(hk, wv_ref[hh], preferred_element_type=F32)
                      + bv_ref[hh])
                sM = lax.dot_general(qh, kh, (((1,), (1,)), ((), ())),
                                     preferred_element_type=F32)
                col = ks + lax.broadcasted_iota(jnp.int32, (BQ, BQ), 1)
                sM = jnp.where((col >= qlo) & (col < qhi), sM, -1e9)
                mnew = jnp.maximum(mx, jnp.max(sM, axis=1, keepdims=True))
                p = jnp.exp(sM - mnew)
                corr = jnp.exp(mx - mnew)
                l2 = l * corr + jnp.sum(p, axis=1, keepdims=True)
                acc2 = acc * corr + jnp.dot(p, vh, preferred_element_type=F32)
                return (mnew, l2, acc2)

            init = (jnp.full((BQ, 1), -1e30, F32), jnp.zeros((BQ, 1), F32),
                    jnp.zeros((BQ, DH), F32))
            mx, l, acc = lax.fori_loop(j0, j1, kblock, init)
            o = o + jnp.dot(acc / l, wo_ref[hh], preferred_element_type=F32)
        out_ref[pl.ds(r0, BQ), :] = o + bo_ref[...]
        return carry

    lax.fori_loop(0, NBQ, qblock, 0)


def _combine_body(h_ref, h1_ref, o_ref, g2_ref, b2_ref, w1_ref, c1_ref,
                  w2_ref, c2_ref, g3_ref, b3_ref, out_ref):
    h2 = _bn_rows(o_ref[...] + h_ref[...], g2_ref[...], b2_ref[...])
    out = h1_ref[...] + h2
    m = jnp.maximum(jnp.dot(out, w1_ref[...], preferred_element_type=F32)
                    + c1_ref[...], 0.0)
    m = jnp.dot(m, w2_ref[...], preferred_element_type=F32) + c2_ref[...]
    out_ref[...] = _bn_rows(out + m, g3_ref[...], b3_ref[...])


def _final_body(h_ref, brow_ref, w1_ref, b1_ref, w2_ref, b2_ref, w3_ref,
                b3_ref, out_ref):
    gm = (lax.broadcasted_iota(jnp.int32, (G, 1), 0) ==
          brow_ref[...]).astype(F32)
    g = jnp.dot(gm, h_ref[...], preferred_element_type=F32)
    r = jnp.maximum(jnp.dot(g, w1_ref[...], preferred_element_type=F32)
                    + b1_ref[...], 0.0)
    r = jnp.maximum(jnp.dot(r, w2_ref[...], preferred_element_type=F32)
                    + b2_ref[...], 0.0)
    out_ref[...] = (jnp.dot(r, w3_ref[...], preferred_element_type=F32)
                    + b3_ref[...])


_TC_PARAMS = pltpu.CompilerParams(vmem_limit_bytes=128 * 1024 * 1024)


def _tc_call(body, n_in, out_shape, smem_args=(), scratch_shapes=()):
    in_specs = [pl.BlockSpec() for _ in range(n_in)]
    for i in smem_args:
        in_specs[i] = pl.BlockSpec(memory_space=pltpu.SMEM)
    return pl.pallas_call(
        body,
        out_shape=jax.ShapeDtypeStruct(out_shape, F32),
        in_specs=in_specs,
        scratch_shapes=list(scratch_shapes),
        compiler_params=_TC_PARAMS)


def kernel(x, pe, edge_index, edge_attr, batch, params):
    p = params
    xcol = x.reshape(N, 1).astype(jnp.int32)
    src = edge_index[0].astype(jnp.int32)
    dst = edge_index[1].astype(jnp.int32)
    attr = edge_attr.astype(jnp.int32)
    b32 = batch.astype(jnp.int32)
    bcol = b32.reshape(N, 1)
    brow = b32.reshape(1, N)

    starts = b32[::BQ]
    ends = b32[BQ - 1::BQ]
    lo = jnp.searchsorted(b32, starts, side="left").astype(jnp.int32)
    hi = jnp.searchsorted(b32, ends, side="right").astype(jnp.int32)
    qlo = jnp.searchsorted(b32, b32, side="left").astype(jnp.int32)
    qhi = jnp.searchsorted(b32, b32, side="right").astype(jnp.int32)
    qlo = qlo.reshape(N, 1)
    qhi = qhi.reshape(N, 1)

    wn = jnp.concatenate([p["node_emb"], jnp.zeros((28, PE), F32)], axis=1)
    wp = jnp.concatenate([jnp.zeros((WL, C - PE), F32), p["pe_lin_W"]],
                         axis=1)
    bc = jnp.concatenate([jnp.zeros((C - PE,), F32),
                          p["pe_lin_b"]]).reshape(1, C)

    embed = _tc_call(_embed_body, 7, (N, C))
    h = embed(xcol, pe, wn, wp, bc, p["pe_norm_g"].reshape(1, WL),
              p["pe_norm_b"].reshape(1, WL))

    zeros = jnp.zeros((N, C), F32)
    gine = _tc_call(_gine_body, 8, (N, C))
    attn = _tc_call(_attn_body, 13, (N, C), smem_args=(3, 4))
    scale = 1.0 / (DH ** 0.5)
    comb = _tc_call(_combine_body, 11, (N, C))

    for lp in p["layers"]:
        ap = _sc_aggr(h, src, dst, attr, p["edge_emb"], zeros)
        h1 = gine(h, ap, lp["gW1"], lp["gb1"].reshape(1, C), lp["gW2"],
                  lp["gb2"].reshape(1, C), lp["n1g"].reshape(1, C),
                  lp["n1b"].reshape(1, C))
        wqkv = lp["Wqkv"]
        bqkv = lp["bqkv"]
        wq3 = wqkv[:, :C].reshape(C, H, DH).transpose(1, 0, 2) * scale
        wk3 = wqkv[:, C:2 * C].reshape(C, H, DH).transpose(1, 0, 2)
        wv3 = wqkv[:, 2 * C:].reshape(C, H, DH).transpose(1, 0, 2)
        bq3 = bqkv[:C].reshape(H, 1, DH) * scale
        bk3 = bqkv[C:2 * C].reshape(H, 1, DH)
        bv3 = bqkv[2 * C:].reshape(H, 1, DH)
        wo3 = lp["Wo"].reshape(H, DH, C)
        o = attn(h, qlo, qhi, lo, hi, wq3, wk3, wv3, bq3, bk3, bv3, wo3,
                 lp["bo"].reshape(1, C))
        h = comb(h, h1, o, lp["n2g"].reshape(1, C), lp["n2b"].reshape(1, C),
                 lp["mW1"], lp["mb1"].reshape(1, 2 * C), lp["mW2"],
                 lp["mb2"].reshape(1, C), lp["n3g"].reshape(1, C),
                 lp["n3b"].reshape(1, C))

    final = _tc_call(_final_body, 8, (G, 1))
    return final(h, brow, p["f_W1"], p["f_b1"].reshape(1, C // 2),
                 p["f_W2"], p["f_b2"].reshape(1, C // 4), p["f_W3"],
                 p["f_b3"].reshape(1, 1))


# speedup vs baseline: 8.4348x; 2.3778x over previous
import functools

import jax
import jax.numpy as jnp
from jax import lax
from jax.experimental import pallas as pl
from jax.experimental.pallas import tpu as pltpu
from jax.experimental.pallas import tpu_sc as plsc

N = 10000
C = 128
G = 64
PE = 8
WL = 20
H = 4
DH = C // H
E = 320000
NINV = 1.0 / N
EPS = 1e-5
F32 = jnp.float32

NC = 2
NS = 16
NW = NC * NS
CH = 128
NCHUNK = E // CH
CPT = (NCHUNK + NW - 1) // NW
RA = 624
RLAST = N - RA * (NS - 1)

BQ = 400
NBQ = N // BQ


def _sc_aggr_body(r_hbm, gidx_hbm, dst_hbm, zeros_hbm, out_hbm,
                  gidx_a, dst_a, gidx_b, dst_b, rows_a, rows_b,
                  sem_a, sem_b, aggr_sh):
    c = lax.axis_index("c")
    s = lax.axis_index("s")
    wid = s * NC + c

    @pl.when(s < NS - 1)
    def _():
        pltpu.sync_copy(zeros_hbm.at[pl.ds(s * RA, RA)],
                        aggr_sh.at[pl.ds(s * RA, RA)])

    @pl.when(s == NS - 1)
    def _():
        pltpu.sync_copy(zeros_hbm.at[pl.ds(RA * (NS - 1), RLAST)],
                        aggr_sh.at[pl.ds(RA * (NS - 1), RLAST)])

    plsc.subcore_barrier()

    slots = ((gidx_a, dst_a, rows_a, sem_a), (gidx_b, dst_b, rows_b, sem_b))

    def fire(m, slot):
        gi, di, rv, sm = slot
        cid = m * NW + wid

        @pl.when(cid < NCHUNK)
        def _():
            base = cid * CH
            pltpu.sync_copy(gidx_hbm.at[pl.ds(base, CH)], gi)
            pltpu.sync_copy(dst_hbm.at[pl.ds(base, CH)], di)
            pltpu.async_copy(r_hbm.at[gi], rv, sm)

    def drain(m, slot):
        gi, di, rv, sm = slot

        @pl.when(m * NW + wid < NCHUNK)
        def _():
            pltpu.make_async_copy(r_hbm.at[pl.ds(0, CH)], rv, sm).wait()
            pltpu.sync_copy(rv, aggr_sh.at[di], add=True)

    fire(0, slots[0])

    def pair_body(t, carry):
        for b in range(2):
            m = t * 2 + b
            fire(m + 1, slots[1 - b])
            drain(m, slots[b])
        return carry

    lax.fori_loop(0, (CPT + 1) // 2, pair_body, 0)
    plsc.subcore_barrier()

    @pl.when(s < NS - 1)
    def _():
        pltpu.sync_copy(aggr_sh.at[pl.ds(s * RA, RA)],
                        out_hbm.at[c, pl.ds(s * RA, RA)])

    @pl.when(s == NS - 1)
    def _():
        pltpu.sync_copy(aggr_sh.at[pl.ds(RA * (NS - 1), RLAST)],
                        out_hbm.at[c, pl.ds(RA * (NS - 1), RLAST)])


def _sc_aggr(r, gidx, dst, zeros):
    mesh = plsc.VectorSubcoreMesh(core_axis_name="c", subcore_axis_name="s",
                                  num_cores=NC, num_subcores=NS)
    f = pl.kernel(
        _sc_aggr_body,
        out_type=jax.ShapeDtypeStruct((NC, N, C), F32),
        mesh=mesh,
        scratch_types=[
            pltpu.VMEM((CH,), jnp.int32),
            pltpu.VMEM((CH,), jnp.int32),
            pltpu.VMEM((CH,), jnp.int32),
            pltpu.VMEM((CH,), jnp.int32),
            pltpu.VMEM((CH, C), F32),
            pltpu.VMEM((CH, C), F32),
            pltpu.SemaphoreType.DMA,
            pltpu.SemaphoreType.DMA,
            pltpu.VMEM_SHARED((N, C), F32),
        ],
    )
    return f(r, gidx, dst, zeros)


def _bn_rows(x, g, b):
    m = jnp.sum(x, axis=0, keepdims=True) * NINV
    ex2 = jnp.sum(x * x, axis=0, keepdims=True) * NINV
    v = ex2 - m * m
    return (x - m) * lax.rsqrt(v + EPS) * g + b


def _embed_body(xcol_ref, pe_ref, wn_ref, wp_ref, bc_ref, g_ref, b_ref,
                out_ref):
    pe = pe_ref[...]
    pen = _bn_rows(pe, g_ref[...], b_ref[...])
    onehot = (xcol_ref[...] ==
              lax.broadcasted_iota(jnp.int32, (1, 28), 1)).astype(F32)
    out_ref[...] = (
        jnp.dot(onehot, wn_ref[...], preferred_element_type=F32)
        + jnp.dot(pen, wp_ref[...], preferred_element_type=F32)
        + bc_ref[...])


def _expand_body(h_ref, ea_ref, out_ref):
    h = h_ref[...]
    for a in range(4):
        out_ref[pl.ds(a * N, N), :] = jnp.maximum(h + ea_ref[a], 0.0)


def _gine_body(h_ref, ap_ref, w1_ref, b1_ref, w2_ref, b2_ref, g_ref, bb_ref,
               out_ref):
    h = h_ref[...]
    z = h + ap_ref[0] + ap_ref[1]
    z = jnp.maximum(jnp.dot(z, w1_ref[...], preferred_element_type=F32)
                    + b1_ref[...], 0.0)
    z = jnp.dot(z, w2_ref[...], preferred_element_type=F32) + b2_ref[...]
    out_ref[...] = _bn_rows(z + h, g_ref[...], bb_ref[...])


def _attn_body(h_ref, qlo_ref, qhi_ref, lo_ref, hi_ref, wq_ref, wk_ref,
               wv_ref, bq_ref, bk_ref, bv_ref, wo_ref, bo_ref, out_ref):
    def qblock(i, carry):
        r0 = i * BQ
        hq = h_ref[pl.ds(r0, BQ), :]
        qlo = qlo_ref[pl.ds(r0, BQ), :]
        qhi = qhi_ref[pl.ds(r0, BQ), :]
        j0 = lo_ref[i] // BQ
        j1 = (hi_ref[i] + BQ - 1) // BQ
        o = jnp.zeros((BQ, C), F32)
        for hh in range(H):
            qh = (jnp.dot(hq, wq_ref[hh], preferred_element_type=F32)
                  + bq_ref[hh])

            def kblock(j, ca):
                mx, l, acc = ca
                ks = j * BQ
                hk = h_ref[pl.ds(ks, BQ), :]
                kh = (jnp.dot(hk, wk_ref[hh], preferred_element_type=F32)
                      + bk_ref[hh])
                vh = (jnp.dot(hk, wv_ref[hh], preferred_element_type=F32)
                      + bv_ref[hh])
                sM = lax.dot_general(qh, kh, (((1,), (1,)), ((), ())),
                                     preferred_element_type=F32)
                col = ks + lax.broadcasted_iota(jnp.int32, (BQ, BQ), 1)
                sM = jnp.where((col >= qlo) & (col < qhi), sM, -1e9)
                mnew = jnp.maximum(mx, jnp.max(sM, axis=1, keepdims=True))
                p = jnp.exp(sM - mnew)
                corr = jnp.exp(mx - mnew)
                l2 = l * corr + jnp.sum(p, axis=1, keepdims=True)
                acc2 = acc * corr + jnp.dot(p, vh, preferred_element_type=F32)
                return (mnew, l2, acc2)

            init = (jnp.full((BQ, 1), -1e30, F32), jnp.zeros((BQ, 1), F32),
                    jnp.zeros((BQ, DH), F32))
            mx, l, acc = lax.fori_loop(j0, j1, kblock, init)
            o = o + jnp.dot(acc / l, wo_ref[hh], preferred_element_type=F32)
        out_ref[pl.ds(r0, BQ), :] = o + bo_ref[...]
        return carry

    lax.fori_loop(0, NBQ, qblock, 0)


def _combine_body(h_ref, h1_ref, o_ref, g2_ref, b2_ref, w1_ref, c1_ref,
                  w2_ref, c2_ref, g3_ref, b3_ref, out_ref):
    h2 = _bn_rows(o_ref[...] + h_ref[...], g2_ref[...], b2_ref[...])
    out = h1_ref[...] + h2
    m = jnp.maximum(jnp.dot(out, w1_ref[...], preferred_element_type=F32)
                    + c1_ref[...], 0.0)
    m = jnp.dot(m, w2_ref[...], preferred_element_type=F32) + c2_ref[...]
    out_ref[...] = _bn_rows(out + m, g3_ref[...], b3_ref[...])


def _final_body(h_ref, brow_ref, w1_ref, b1_ref, w2_ref, b2_ref, w3_ref,
                b3_ref, out_ref):
    gm = (lax.broadcasted_iota(jnp.int32, (G, 1), 0) ==
          brow_ref[...]).astype(F32)
    g = jnp.dot(gm, h_ref[...], preferred_element_type=F32)
    r = jnp.maximum(jnp.dot(g, w1_ref[...], preferred_element_type=F32)
                    + b1_ref[...], 0.0)
    r = jnp.maximum(jnp.dot(r, w2_ref[...], preferred_element_type=F32)
                    + b2_ref[...], 0.0)
    out_ref[...] = (jnp.dot(r, w3_ref[...], preferred_element_type=F32)
                    + b3_ref[...])


_TC_PARAMS = pltpu.CompilerParams(vmem_limit_bytes=128 * 1024 * 1024)


def _tc_call(body, n_in, out_shape, smem_args=(), scratch_shapes=()):
    in_specs = [pl.BlockSpec() for _ in range(n_in)]
    for i in smem_args:
        in_specs[i] = pl.BlockSpec(memory_space=pltpu.SMEM)
    return pl.pallas_call(
        body,
        out_shape=jax.ShapeDtypeStruct(out_shape, F32),
        in_specs=in_specs,
        scratch_shapes=list(scratch_shapes),
        compiler_params=_TC_PARAMS)


def kernel(x, pe, edge_index, edge_attr, batch, params):
    p = params
    xcol = x.reshape(N, 1).astype(jnp.int32)
    src = edge_index[0].astype(jnp.int32)
    dst = edge_index[1].astype(jnp.int32)
    attr = edge_attr.astype(jnp.int32)
    b32 = batch.astype(jnp.int32)
    bcol = b32.reshape(N, 1)
    brow = b32.reshape(1, N)

    oneh = (bcol == lax.broadcasted_iota(jnp.int32, (1, G), 1))
    counts = jnp.sum(oneh.astype(jnp.int32), axis=0)
    cum = jnp.cumsum(counts)
    seg_start = cum - counts
    qlo = jnp.sum(jnp.where(oneh, seg_start[None, :], 0), axis=1,
                  dtype=jnp.int32).reshape(N, 1)
    qhi = jnp.sum(jnp.where(oneh, cum[None, :], 0), axis=1,
                  dtype=jnp.int32).reshape(N, 1)
    lo = qlo[::BQ, 0]
    hi = qhi[BQ - 1::BQ, 0]

    wn = jnp.concatenate([p["node_emb"], jnp.zeros((28, PE), F32)], axis=1)
    wp = jnp.concatenate([jnp.zeros((WL, C - PE), F32), p["pe_lin_W"]],
                         axis=1)
    bc = jnp.concatenate([jnp.zeros((C - PE,), F32),
                          p["pe_lin_b"]]).reshape(1, C)

    embed = _tc_call(_embed_body, 7, (N, C))
    h = embed(xcol, pe, wn, wp, bc, p["pe_norm_g"].reshape(1, WL),
              p["pe_norm_b"].reshape(1, WL))

    gidx = attr * N + src
    zeros = jnp.zeros((N, C), F32)
    expand = _tc_call(_expand_body, 2, (4 * N, C))
    gine = _tc_call(_gine_body, 8, (N, C))
    attn = _tc_call(_attn_body, 13, (N, C), smem_args=(3, 4))
    scale = 1.0 / (DH ** 0.5)
    comb = _tc_call(_combine_body, 11, (N, C))

    for lp in p["layers"]:
        r = expand(h, p["edge_emb"])
        ap = _sc_aggr(r, gidx, dst, zeros)
        h1 = gine(h, ap, lp["gW1"], lp["gb1"].reshape(1, C), lp["gW2"],
                  lp["gb2"].reshape(1, C), lp["n1g"].reshape(1, C),
                  lp["n1b"].reshape(1, C))
        wqkv = lp["Wqkv"]
        bqkv = lp["bqkv"]
        wq3 = wqkv[:, :C].reshape(C, H, DH).transpose(1, 0, 2) * scale
        wk3 = wqkv[:, C:2 * C].reshape(C, H, DH).transpose(1, 0, 2)
        wv3 = wqkv[:, 2 * C:].reshape(C, H, DH).transpose(1, 0, 2)
        bq3 = bqkv[:C].reshape(H, 1, DH) * scale
        bk3 = bqkv[C:2 * C].reshape(H, 1, DH)
        bv3 = bqkv[2 * C:].reshape(H, 1, DH)
        wo3 = lp["Wo"].reshape(H, DH, C)
        o = attn(h, qlo, qhi, lo, hi, wq3, wk3, wv3, bq3, bk3, bv3, wo3,
                 lp["bo"].reshape(1, C))
        h = comb(h, h1, o, lp["n2g"].reshape(1, C), lp["n2b"].reshape(1, C),
                 lp["mW1"], lp["mb1"].reshape(1, 2 * C), lp["mW2"],
                 lp["mb2"].reshape(1, C), lp["n3g"].reshape(1, C),
                 lp["n3b"].reshape(1, C))

    final = _tc_call(_final_body, 8, (G, 1))
    return final(h, brow, p["f_W1"], p["f_b1"].reshape(1, C // 2),
                 p["f_W2"], p["f_b2"].reshape(1, C // 4), p["f_W3"],
                 p["f_b3"].reshape(1, 1))
